# baseline (speedup 1.0000x reference)
import jax
import jax.numpy as jnp
from jax import lax
from jax.experimental import pallas as pl
from jax.experimental.pallas import tpu as pltpu

_N = 5000
_NP = 5120
_B = 128
_NB = _NP // _B
_T = 0.5
_MAXD = 100

def _iou_rc(rx1, ry1, rx2, ry2, cx1, cy1, cx2, cy2):
    area_r = (rx2 - rx1) * (ry2 - ry1)
    area_c = (cx2 - cx1) * (cy2 - cy1)
    ltx = jnp.maximum(rx1, cx1)
    lty = jnp.maximum(ry1, cy1)
    rbx = jnp.minimum(rx2, cx2)
    rby = jnp.minimum(ry2, cy2)
    w = jnp.clip(rbx - ltx, 0.0, None)
    h = jnp.clip(rby - lty, 0.0, None)
    inter = w * h
    union = area_r + area_c - inter
    return inter / jnp.maximum(union, 1e-9)


def _nms_body(key_ref, x1_ref, y1_ref, x2_ref, y2_ref, out_ref, keep_ref, post_ref):
    f32 = jnp.float32

    ss2d = -key_ref[:, :]
    keep_ref[:, :] = (ss2d > 0.0).astype(f32)

    riota = lax.broadcasted_iota(jnp.int32, (_B, _B), 0)
    ciota = lax.broadcasted_iota(jnp.int32, (_B, _B), 1)
    tri = (ciota > riota).astype(f32)

    def diag_cond(carry):
        d, count = carry
        return jnp.logical_and(d < _NB, count < _MAXD)

    def diag_body(carry):
        d, count = carry
        o = d * _B
        cx1 = x1_ref[0:1, pl.ds(o, _B)]
        cy1 = y1_ref[0:1, pl.ds(o, _B)]
        cx2 = x2_ref[0:1, pl.ds(o, _B)]
        cy2 = y2_ref[0:1, pl.ds(o, _B)]
        rt = jnp.transpose(
            jnp.concatenate([cx1, cy1, cx2, cy2], axis=0))
        rx1 = rt[:, 0:1]
        ry1 = rt[:, 1:2]
        rx2 = rt[:, 2:3]
        ry2 = rt[:, 3:4]
        iou = _iou_rc(rx1, ry1, rx2, ry2, cx1, cy1, cx2, cy2)
        sf = jnp.where(iou > _T, tri, 0.0)

        k0 = keep_ref[pl.ds(d, 1), :]

        def fcond(c):
            _, changed, it = c
            return jnp.logical_and(changed, it <= _B)

        def fbody(c):
            k, _, it = c
            sup = jnp.dot(k, sf, preferred_element_type=f32)
            knew = jnp.where(sup > 0.5, 0.0, k0)
            return knew, jnp.any(knew != k), it + 1

        kf, _, _ = lax.while_loop(fcond, fbody, (k0, True, 0))
        keep_ref[pl.ds(d, 1), :] = kf
        count = count + jnp.sum(kf).astype(jnp.int32)

        def cbody(c, _):
            oc = c * _B
            ccx1 = x1_ref[0:1, pl.ds(oc, _B)]
            ccy1 = y1_ref[0:1, pl.ds(oc, _B)]
            ccx2 = x2_ref[0:1, pl.ds(oc, _B)]
            ccy2 = y2_ref[0:1, pl.ds(oc, _B)]
            iou_c = _iou_rc(rx1, ry1, rx2, ry2, ccx1, ccy1, ccx2, ccy2)
            sc = (iou_c > _T).astype(f32)
            sup = jnp.dot(kf, sc, preferred_element_type=f32)
            kc = keep_ref[pl.ds(c, 1), :]
            keep_ref[pl.ds(c, 1), :] = jnp.where(sup > 0.5, 0.0, kc)
            return 0

        ub = jnp.where(count < _MAXD, _NB, d + 1)
        lax.fori_loop(d + 1, ub, cbody, 0)
        return d + 1, count

    _, count = lax.while_loop(diag_cond, diag_body, (jnp.int32(0), jnp.int32(0)))

    keep2 = keep_ref[:, :]
    jr = lax.broadcasted_iota(jnp.int32, (_NB, _B), 0)
    jc = lax.broadcasted_iota(jnp.int32, (_NB, _B), 1)
    jidx = jr * _B + jc
    lt_incl = (lax.broadcasted_iota(jnp.int32, (_B, _B), 0)
               <= lax.broadcasted_iota(jnp.int32, (_B, _B), 1)).astype(f32)
    intra = jnp.dot(keep2, lt_incl, preferred_element_type=f32)
    rows = jnp.sum(keep2, axis=1, keepdims=True)
    lt_strict = (lax.broadcasted_iota(jnp.int32, (_NB, _NB), 1)
                 < lax.broadcasted_iota(jnp.int32, (_NB, _NB), 0)).astype(f32)
    rowpfx = jnp.dot(lt_strict, rows, preferred_element_type=f32)
    c1 = intra + rowpfx
    cnt_f = count.astype(f32)
    pos = jnp.where(keep2 > 0.5, c1 - 1.0,
                    cnt_f + jidx.astype(f32) - c1)
    pos = jnp.minimum(pos, 127.0)
    post_ref[:, :] = pos
    prow = lax.broadcasted_iota(jnp.int32, (1, _B), 1).astype(f32)
    minpos = jnp.min(pos, axis=1, keepdims=True)
    tio = lax.broadcasted_iota(jnp.int32, (_NB, 1), 0)
    t_ub = jnp.max(jnp.where(minpos < 99.5, tio, 0)) + 1

    def sel_body(c, acc_t):
        ptile = jnp.transpose(post_ref[pl.ds(c, 1), :])
        m2t = (ptile == prow).astype(f32)
        oc = c * _B
        lhs = jnp.concatenate(
            [x1_ref[0:1, pl.ds(oc, _B)],
             y1_ref[0:1, pl.ds(oc, _B)],
             x2_ref[0:1, pl.ds(oc, _B)],
             y2_ref[0:1, pl.ds(oc, _B)],
             -key_ref[pl.ds(c, 1), :]], axis=0)
        return acc_t + jnp.dot(lhs, m2t,
                               preferred_element_type=f32,
                               precision=lax.Precision.HIGHEST)

    acc_t = lax.fori_loop(0, t_ub, sel_body, jnp.zeros((5, _B), f32))
    acc = jnp.transpose(acc_t)
    piota = lax.broadcasted_iota(jnp.int32, (_B, 1), 0).astype(f32)
    out_ref[:, 0:4] = acc[0:_MAXD, 0:4]
    out_ref[:, 4:5] = (acc[:, 4:5] * (piota < cnt_f))[0:_MAXD, :]
    out_ref[:, 5:8] = jnp.zeros((_MAXD, 3), f32)


def _run_nms(key2d, x1r, y1r, x2r, y2r, interpret=False):
    return pl.pallas_call(
        _nms_body,
        out_shape=jax.ShapeDtypeStruct((_MAXD, 8), jnp.float32),
        scratch_shapes=[
            pltpu.VMEM((_NB, _B), jnp.float32),
            pltpu.VMEM((_NB, _B), jnp.float32),
        ],
        interpret=interpret,
    )(key2d, x1r, y1r, x2r, y2r)


def kernel(boxes, scores):
    s = jnp.where(scores > 0.05, scores, -1.0)
    pad = _NP - _N
    s_p = jnp.concatenate([s, jnp.full((pad,), -1.0, jnp.float32)])
    b_p = jnp.concatenate([boxes, jnp.zeros((pad, 4), jnp.float32)], axis=0)
    srt = lax.sort((-s_p, b_p[:, 0], b_p[:, 1], b_p[:, 2], b_p[:, 3]),
                   num_keys=1, is_stable=True)
    out = _run_nms(srt[0].reshape(_NB, _B),
                   srt[1].reshape(1, _NP), srt[2].reshape(1, _NP),
                   srt[3].reshape(1, _NP), srt[4].reshape(1, _NP))
    return out[:, :5]
